# Initial kernel scaffold; baseline (speedup 1.0000x reference)
import jax
import jax.numpy as jnp
from jax.experimental import pallas as pl


def kernel(nodes, neighbours_full, features):
    raise NotImplementedError("write your pallas kernel here")


# speedup vs baseline: 1.3206x; 1.3206x over previous
import jax
import jax.numpy as jnp
from jax import lax
from jax.experimental import pallas as pl
from jax.experimental.pallas import tpu as pltpu
from jax.experimental.pallas import tpu_sc as plsc

B = 16384
D = 128
S = 11
NUM_SAMPLE = 10
NW = 32
GROUP = 128
G_PER_W = B // (NW * GROUP)
B_PER_W = G_PER_W * GROUP
STEPS = S * G_PER_W
LANES = 16
INV = 1.0 / S


def _agg_body(ids_hbm, feat_hbm, out_hbm, idx_v, rows0, rows1, acc, sem0, sem1):
  wid = lax.axis_index("s") * 2 + lax.axis_index("c")
  gbase = wid * G_PER_W

  for j in range(S):
    pltpu.sync_copy(ids_hbm.at[j, pl.ds(gbase, G_PER_W)], idx_v.at[j])

  zero = jnp.zeros((LANES,), jnp.float32)

  def zbody(r, _):
    for c in range(D // LANES):
      acc[r, pl.ds(c * LANES, LANES)] = zero
    return 0

  lax.fori_loop(0, B_PER_W, zbody, 0)

  rows = (rows0, rows1)
  sems = (sem0, sem1)

  def issue(step, b):
    j = step // G_PER_W
    g = lax.rem(step, G_PER_W)
    pltpu.async_copy(feat_hbm.at[idx_v.at[j, g]], rows[b], sems[b])

  def drain(b):
    pltpu.make_async_copy(feat_hbm.at[idx_v.at[0, 0]], rows[b], sems[b]).wait()

  issue(jnp.int32(0), 0)
  issue(jnp.int32(1), 1)

  def accum(step, b):
    g = lax.rem(step, G_PER_W)
    base_row = g * GROUP

    def abody(r, _):
      row = base_row + r
      for c in range(D // LANES):
        sl = pl.ds(c * LANES, LANES)
        plsc.addupdate(acc.at[row, sl], rows[b][r, sl] * INV)
      return 0

    lax.fori_loop(0, GROUP, abody, 0)

  def body(it, _):
    for b in range(2):
      step = it * 2 + b
      drain(b)
      accum(step, b)
      nxt = step + 2

      @pl.when(nxt < STEPS)
      def _():
        issue(nxt, b)

    return 0

  lax.fori_loop(0, STEPS // 2, body, 0)

  pltpu.sync_copy(acc, out_hbm.at[pl.ds(wid * B_PER_W, B_PER_W)])


@jax.jit
def _agg(ids_r, features):
  mesh = plsc.VectorSubcoreMesh(core_axis_name="c", subcore_axis_name="s")
  return pl.kernel(
      _agg_body,
      out_type=jax.ShapeDtypeStruct((B, D), jnp.float32),
      mesh=mesh,
      scratch_types=[
          pltpu.VMEM((S, G_PER_W, GROUP), jnp.int32),
          pltpu.VMEM((GROUP, D), jnp.float32),
          pltpu.VMEM((GROUP, D), jnp.float32),
          pltpu.VMEM((B_PER_W, D), jnp.float32),
          pltpu.SemaphoreType.DMA,
          pltpu.SemaphoreType.DMA,
      ],
  )(ids_r, features)


def kernel(nodes, neighbours_full, features):
  all_ids = jnp.concatenate(
      [neighbours_full[:, :NUM_SAMPLE], nodes[:, None]], axis=1)
  ids_r = all_ids.T.reshape(S, B // GROUP, GROUP)
  return _agg(ids_r, features)


# speedup vs baseline: 3.7132x; 2.8117x over previous
import jax
import jax.numpy as jnp
from jax import lax
from jax.experimental import pallas as pl
from jax.experimental.pallas import tpu as pltpu
from jax.experimental.pallas import tpu_sc as plsc

B = 16384
D = 128
S = 11
NUM_SAMPLE = 10
NW = 32
GROUP = 128
G_PER_W = B // (NW * GROUP)
B_PER_W = G_PER_W * GROUP
HALF = 2 * GROUP
ACC_ROWS = 16 * HALF
LANES = 16
INV = 1.0 / S


def _agg_body(ids_hbm, feat_hbm, out_hbm, idx_v, scat, rows, sems, shared):
  cid = lax.axis_index("c")
  sid = lax.axis_index("s")
  wid = sid * 2 + cid
  gbase = wid * G_PER_W
  lbase = sid * HALF

  for j in range(S):
    pltpu.sync_copy(ids_hbm.at[j, pl.ds(gbase, G_PER_W)], idx_v.at[j])

  iota = lax.iota(jnp.int32, LANES)
  for k in range(2):
    for c in range(D // LANES):
      scat[k, pl.ds(c * LANES, LANES)] = lbase + k * GROUP + c * LANES + iota

  def issue(j, g, b):
    pltpu.async_copy(feat_hbm.at[idx_v.at[j, g]], rows[b], sems[b])

  def drain(b):
    pltpu.make_async_copy(feat_hbm.at[idx_v.at[0, 0]], rows[b], sems[b]).wait()

  def half_body(h, _):
    g0 = h * 2
    g1 = h * 2 + 1

    issue(jnp.int32(0), g0, 0)
    issue(jnp.int32(0), g1, 1)
    issue(jnp.int32(1), g0, 2)
    issue(jnp.int32(1), g1, 3)

    drain(0)
    pltpu.sync_copy(rows[0], shared.at[pl.ds(lbase, GROUP)])
    issue(jnp.int32(2), g0, 0)
    drain(1)
    pltpu.sync_copy(rows[1], shared.at[pl.ds(lbase + GROUP, GROUP)])
    issue(jnp.int32(2), g1, 1)

    def tbody(t, _):
      ja = 2 * t + 3
      jb = 2 * t + 4

      drain(2)
      pltpu.sync_copy(rows[2], shared.at[scat.at[0]], add=True)

      @pl.when(ja < S)
      def _():
        issue(ja, g0, 2)

      drain(3)
      pltpu.sync_copy(rows[3], shared.at[scat.at[1]], add=True)

      @pl.when(ja < S)
      def _():
        issue(ja, g1, 3)

      drain(0)
      pltpu.sync_copy(rows[0], shared.at[scat.at[0]], add=True)

      @pl.when(jb < S)
      def _():
        issue(jb, g0, 0)

      drain(1)
      pltpu.sync_copy(rows[1], shared.at[scat.at[1]], add=True)

      @pl.when(jb < S)
      def _():
        issue(jb, g1, 1)

      return 0

    lax.fori_loop(0, 5, tbody, 0)

    obase = wid * B_PER_W + h * HALF
    for k in range(2):
      pltpu.sync_copy(shared.at[pl.ds(lbase + k * GROUP, GROUP)], rows[k])

      def sbody(r, _, k=k):
        for c in range(D // LANES):
          sl = pl.ds(c * LANES, LANES)
          rows[k][r, sl] = rows[k][r, sl] * INV
        return 0

      lax.fori_loop(0, GROUP, sbody, 0)
      pltpu.async_copy(
          rows[k], out_hbm.at[pl.ds(obase + k * GROUP, GROUP)], sems[k])

    for k in range(2):
      pltpu.make_async_copy(
          rows[k], out_hbm.at[pl.ds(obase + k * GROUP, GROUP)],
          sems[k]).wait()
    return 0

  lax.fori_loop(0, 2, half_body, 0)


@jax.jit
def _agg(ids_r, features):
  mesh = plsc.VectorSubcoreMesh(core_axis_name="c", subcore_axis_name="s")
  return pl.kernel(
      _agg_body,
      out_type=jax.ShapeDtypeStruct((B, D), jnp.float32),
      mesh=mesh,
      scratch_types=[
          pltpu.VMEM((S, G_PER_W, GROUP), jnp.int32),
          pltpu.VMEM((2, GROUP), jnp.int32),
          [pltpu.VMEM((GROUP, D), jnp.float32)] * 4,
          [pltpu.SemaphoreType.DMA] * 4,
          pltpu.VMEM_SHARED((ACC_ROWS, D), jnp.float32),
      ],
  )(ids_r, features)


def kernel(nodes, neighbours_full, features):
  all_ids = jnp.concatenate(
      [nodes[:, None], neighbours_full[:, :NUM_SAMPLE]], axis=1)
  ids_r = all_ids.T.reshape(S, B // GROUP, GROUP)
  return _agg(ids_r, features)
